# Initial kernel scaffold; baseline (speedup 1.0000x reference)
import jax
import jax.numpy as jnp
from jax.experimental import pallas as pl


def kernel(W1, b1, W2, b2, W3, b3):
    raise NotImplementedError("write your pallas kernel here")


# speedup vs baseline: 37.5279x; 37.5279x over previous
import functools

import numpy as np
import jax
import jax.numpy as jnp
from jax import lax
from jax.experimental import pallas as pl
from jax.experimental.pallas import tpu as pltpu
from jax.experimental.pallas import tpu_sc as plsc

COUNT = 255
BATCH = 8
NODES_PER_TILE = 8
COLS_PER_TILE = 40
NUM_CORES = 2
NUM_SUBCORES = 16
NUM_TILES = 32
TASKS = 2048
KMAX = 6


_ROT = [[13, 15, 26, 6], [17, 29, 16, 24]]


def _rotl(x, d):
    return ((x << np.uint32(d)) | (x >> np.uint32(32 - d))).astype(np.uint32)


def _threefry2x32(k1, k2, x0, x1):
    ks = [np.uint32(k1), np.uint32(k2), np.uint32(k1 ^ k2 ^ 0x1BD11BDA)]
    x = [(x0 + ks[0]).astype(np.uint32), (x1 + ks[1]).astype(np.uint32)]

    def rounds(x, rots):
        for r in rots:
            a = (x[0] + x[1]).astype(np.uint32)
            b = (_rotl(x[1], r) ^ a).astype(np.uint32)
            x = [a, b]
        return x

    for i, (inj0, inj1) in enumerate([(1, 2), (2, 0), (0, 1), (1, 2), (2, 0)]):
        x = rounds(x, _ROT[i % 2])
        x = [(x[0] + ks[inj0]).astype(np.uint32),
             (x[1] + ks[inj1] + np.uint32(i + 1)).astype(np.uint32)]
    return x[0], x[1]


def _fold_in(key, i):
    y0, y1 = _threefry2x32(key[0], key[1],
                           np.array([0], np.uint32), np.array([i], np.uint32))
    return (int(y0[0]), int(y1[0]))


def _gumbel(key, shape):
    n = int(np.prod(shape))
    y0, y1 = _threefry2x32(key[0], key[1],
                           np.zeros(n, np.uint32), np.arange(n, dtype=np.uint32))
    bits = y0 ^ y1
    u = ((bits >> np.uint32(9)) | np.uint32(0x3F800000)).view(np.float32) - np.float32(1.0)
    tiny = np.float32(np.finfo(np.float32).tiny)
    u = np.maximum(tiny, (u * (np.float32(1.0) - tiny) + tiny).astype(np.float32))
    return (-np.log(-np.log(u.astype(np.float64)))).astype(np.float32).reshape(shape)


def _build_gumbel_table():
    g = np.zeros((KMAX, TASKS), np.float32)
    for n in range(COUNT):
        arity = 6 if n % 2 == 0 else 4
        gn = _gumbel(_fold_in((0, 42), n), (BATCH, arity))
        g[:arity, n * BATCH:(n + 1) * BATCH] = gn.T
    return g


_GUMBEL = _build_gumbel_table()


_NEG = -3.0e38


def _sc_body(w2t_hbm, w3t_hbm, b1_hbm, gum_hbm, act_hbm, sel_hbm,
             w2t_v, w3t_v, b1_v, gum_v, logits_v, act_v, sel_v):
    wid = lax.axis_index("s") * NUM_CORES + lax.axis_index("c")

    pltpu.sync_copy(b1_hbm, b1_v)
    pltpu.sync_copy(w2t_hbm, w2t_v)
    pltpu.sync_copy(w3t_hbm.at[pl.ds(wid * (COLS_PER_TILE * 64), COLS_PER_TILE * 64)],
                    w3t_v)
    for k in range(KMAX):
        pltpu.sync_copy(gum_hbm.at[pl.ds(k * TASKS + wid * 64, 64)],
                        gum_v.at[pl.ds(k * 64, 64)])

    h1 = [jnp.maximum(b1_v[pl.ds(16 * q, 16)], 0.0) for q in range(4)]
    h2 = []
    for j in range(60):
        acc = h1[0] * w2t_v[pl.ds(j * 64, 16)]
        for q in range(1, 4):
            acc = acc + h1[q] * w2t_v[pl.ds(j * 64 + 16 * q, 16)]
        h2.append(jnp.maximum(jnp.sum(acc), 0.0))

    lane = lax.iota(jnp.int32, 16)
    for q in range(3):
        r = jnp.minimum(lane + 16 * q, COLS_PER_TILE - 1)
        base = r * 64
        acc = plsc.load_gather(w3t_v, [base + 60])
        for k in range(60):
            acc = acc + h2[k] * plsc.load_gather(w3t_v, [base + k])
        e = jnp.exp(acc * 0.4)
        logits_v[pl.ds(16 * q, 16)] = 2.5 * (1.0 - 2.0 / (e + 1.0))

    is_hi = lane >= 8
    for g in range(4):
        colstart = jnp.where(is_hi, 10 * g + 6, 10 * g)
        arity = jnp.where(is_hi, 4, 6)
        vk = [plsc.load_gather(logits_v, [colstart + k]) for k in range(KMAX)]
        gk = [gum_v[pl.ds(k * 64 + 16 * g, 16)] for k in range(KMAX)]
        valid = [arity > k for k in range(KMAX)]
        m = jnp.full((16,), _NEG, jnp.float32)
        for k in range(KMAX):
            m = jnp.maximum(m, jnp.where(valid[k], vk[k], _NEG))
        sumexp = jnp.zeros((16,), jnp.float32)
        best_s = jnp.full((16,), _NEG, jnp.float32)
        best_k = jnp.zeros((16,), jnp.int32)
        best_e = jnp.zeros((16,), jnp.float32)
        for k in range(KMAX):
            e = jnp.exp(vk[k] - m)
            sumexp = sumexp + jnp.where(valid[k], e, 0.0)
            s = jnp.where(valid[k], vk[k] + gk[k], _NEG)
            upd = s > best_s
            best_s = jnp.where(upd, s, best_s)
            best_k = jnp.where(upd, jnp.full((16,), k, jnp.int32), best_k)
            best_e = jnp.where(upd, e, best_e)
        act_v[pl.ds(16 * g, 16)] = best_k
        sel_v[pl.ds(16 * g, 16)] = best_e / sumexp

    pltpu.sync_copy(act_v, act_hbm.at[pl.ds(wid * 64, 64)])
    pltpu.sync_copy(sel_v, sel_hbm.at[pl.ds(wid * 64, 64)])


@functools.cache
def _sc_kernel():
    return pl.kernel(
        _sc_body,
        out_type=(jax.ShapeDtypeStruct((TASKS,), jnp.int32),
                  jax.ShapeDtypeStruct((TASKS,), jnp.float32)),
        mesh=plsc.VectorSubcoreMesh(core_axis_name="c", subcore_axis_name="s",
                                    num_cores=NUM_CORES,
                                    num_subcores=NUM_SUBCORES),
        compiler_params=pltpu.CompilerParams(needs_layout_passes=False),
        scratch_types=[
            pltpu.VMEM((60 * 64,), jnp.float32),
            pltpu.VMEM((COLS_PER_TILE * 64,), jnp.float32),
            pltpu.VMEM((64,), jnp.float32),
            pltpu.VMEM((KMAX * 64,), jnp.float32),
            pltpu.VMEM((48,), jnp.float32),
            pltpu.VMEM((64,), jnp.int32),
            pltpu.VMEM((64,), jnp.float32),
        ],
    )


def kernel(W1, b1, W2, b2, W3, b3):
    f32 = jnp.float32
    b1p = jnp.concatenate([b1, jnp.array([1.0, 0.0, 0.0, 0.0], f32)])
    w2t = jnp.concatenate(
        [W2.T, b2[:, None], jnp.zeros((60, 3), f32)], axis=1).reshape(-1)
    w3t = jnp.concatenate(
        [W3.T, b3[:, None], jnp.zeros((W3.shape[1], 3), f32)], axis=1)
    w3t = jnp.concatenate([w3t, jnp.zeros((NUM_TILES * COLS_PER_TILE - W3.shape[1], 64),
                                          f32)], axis=0).reshape(-1)
    gum = jnp.asarray(_GUMBEL).reshape(-1)

    act_flat, sel_flat = _sc_kernel()(w2t, w3t, b1p, gum)
    actions = act_flat[:COUNT * BATCH].reshape(COUNT, BATCH).T
    selected = sel_flat[:COUNT * BATCH].reshape(COUNT, BATCH).T
    return (actions, selected)


# speedup vs baseline: 42.1723x; 1.1238x over previous
import functools

import numpy as np
import jax
import jax.numpy as jnp
from jax import lax
from jax.experimental import pallas as pl
from jax.experimental.pallas import tpu as pltpu
from jax.experimental.pallas import tpu_sc as plsc

COUNT = 255
BATCH = 8
NODES_PER_TILE = 8
COLS_PER_TILE = 40
NUM_CORES = 2
NUM_SUBCORES = 16
NUM_TILES = 32
TASKS = 2048
KMAX = 6


_ROT = [[13, 15, 26, 6], [17, 29, 16, 24]]


def _rotl(x, d):
    return ((x << np.uint32(d)) | (x >> np.uint32(32 - d))).astype(np.uint32)


def _threefry2x32(k1, k2, x0, x1):
    ks = [np.uint32(k1), np.uint32(k2), np.uint32(k1 ^ k2 ^ 0x1BD11BDA)]
    x = [(x0 + ks[0]).astype(np.uint32), (x1 + ks[1]).astype(np.uint32)]

    def rounds(x, rots):
        for r in rots:
            a = (x[0] + x[1]).astype(np.uint32)
            b = (_rotl(x[1], r) ^ a).astype(np.uint32)
            x = [a, b]
        return x

    for i, (inj0, inj1) in enumerate([(1, 2), (2, 0), (0, 1), (1, 2), (2, 0)]):
        x = rounds(x, _ROT[i % 2])
        x = [(x[0] + ks[inj0]).astype(np.uint32),
             (x[1] + ks[inj1] + np.uint32(i + 1)).astype(np.uint32)]
    return x[0], x[1]


def _fold_in(key, i):
    y0, y1 = _threefry2x32(key[0], key[1],
                           np.array([0], np.uint32), np.array([i], np.uint32))
    return (int(y0[0]), int(y1[0]))


def _gumbel(key, shape):
    n = int(np.prod(shape))
    y0, y1 = _threefry2x32(key[0], key[1],
                           np.zeros(n, np.uint32), np.arange(n, dtype=np.uint32))
    bits = y0 ^ y1
    u = ((bits >> np.uint32(9)) | np.uint32(0x3F800000)).view(np.float32) - np.float32(1.0)
    tiny = np.float32(np.finfo(np.float32).tiny)
    u = np.maximum(tiny, (u * (np.float32(1.0) - tiny) + tiny).astype(np.float32))
    return (-np.log(-np.log(u.astype(np.float64)))).astype(np.float32).reshape(shape)


def _build_gumbel_table():
    g = np.zeros((KMAX, TASKS), np.float32)
    for n in range(COUNT):
        arity = 6 if n % 2 == 0 else 4
        gn = _gumbel(_fold_in((0, 42), n), (BATCH, arity))
        g[:arity, n * BATCH:(n + 1) * BATCH] = gn.T
    return g


_GUMBEL = _build_gumbel_table()
_GUMBEL_TILED = np.ascontiguousarray(
    _GUMBEL.reshape(KMAX, 32, 64).transpose(1, 0, 2)).reshape(-1)


_NEG = -3.0e38


def _sc_body(w2t_hbm, w3t_hbm, b1_hbm, gum_hbm, act_hbm, sel_hbm,
             w2t_v, w3t_v, b1_v, gum_v, logits_v, act_v, sel_v, sem):
    wid = lax.axis_index("s") * NUM_CORES + lax.axis_index("c")

    copies = [
        pltpu.async_copy(b1_hbm, b1_v, sem),
        pltpu.async_copy(w2t_hbm, w2t_v, sem),
        pltpu.async_copy(
            w3t_hbm.at[pl.ds(wid * (COLS_PER_TILE * 64), COLS_PER_TILE * 64)],
            w3t_v, sem),
        pltpu.async_copy(gum_hbm.at[pl.ds(wid * (KMAX * 64), KMAX * 64)],
                         gum_v, sem),
    ]
    for c in copies:
        c.wait()

    h1 = [jnp.maximum(b1_v[pl.ds(16 * q, 16)], 0.0) for q in range(4)]
    h2 = []
    for j in range(60):
        acc = h1[0] * w2t_v[pl.ds(j * 64, 16)]
        for q in range(1, 4):
            acc = acc + h1[q] * w2t_v[pl.ds(j * 64 + 16 * q, 16)]
        h2.append(jnp.maximum(jnp.sum(acc), 0.0))

    lane = lax.iota(jnp.int32, 16)
    for q in range(3):
        r = jnp.minimum(lane + 16 * q, COLS_PER_TILE - 1)
        base = r * 64
        acc = plsc.load_gather(w3t_v, [base + 60])
        for k in range(60):
            acc = acc + h2[k] * plsc.load_gather(w3t_v, [base + k])
        e = jnp.exp(acc * 0.4)
        logits_v[pl.ds(16 * q, 16)] = 2.5 * (1.0 - 2.0 / (e + 1.0))

    is_hi = lane >= 8
    for g in range(4):
        colstart = jnp.where(is_hi, 10 * g + 6, 10 * g)
        arity = jnp.where(is_hi, 4, 6)
        vk = [plsc.load_gather(logits_v, [colstart + k]) for k in range(KMAX)]
        gk = [gum_v[pl.ds(k * 64 + 16 * g, 16)] for k in range(KMAX)]
        valid = [arity > k for k in range(KMAX)]
        m = jnp.full((16,), _NEG, jnp.float32)
        for k in range(KMAX):
            m = jnp.maximum(m, jnp.where(valid[k], vk[k], _NEG))
        sumexp = jnp.zeros((16,), jnp.float32)
        best_s = jnp.full((16,), _NEG, jnp.float32)
        best_k = jnp.zeros((16,), jnp.int32)
        best_e = jnp.zeros((16,), jnp.float32)
        for k in range(KMAX):
            e = jnp.exp(vk[k] - m)
            sumexp = sumexp + jnp.where(valid[k], e, 0.0)
            s = jnp.where(valid[k], vk[k] + gk[k], _NEG)
            upd = s > best_s
            best_s = jnp.where(upd, s, best_s)
            best_k = jnp.where(upd, jnp.full((16,), k, jnp.int32), best_k)
            best_e = jnp.where(upd, e, best_e)
        act_v[pl.ds(16 * g, 16)] = best_k
        sel_v[pl.ds(16 * g, 16)] = best_e / sumexp

    outs = [
        pltpu.async_copy(act_v, act_hbm.at[pl.ds(wid * 64, 64)], sem),
        pltpu.async_copy(sel_v, sel_hbm.at[pl.ds(wid * 64, 64)], sem),
    ]
    for c in outs:
        c.wait()


@functools.cache
def _sc_kernel():
    return pl.kernel(
        _sc_body,
        out_type=(jax.ShapeDtypeStruct((TASKS,), jnp.int32),
                  jax.ShapeDtypeStruct((TASKS,), jnp.float32)),
        mesh=plsc.VectorSubcoreMesh(core_axis_name="c", subcore_axis_name="s",
                                    num_cores=NUM_CORES,
                                    num_subcores=NUM_SUBCORES),
        compiler_params=pltpu.CompilerParams(needs_layout_passes=False),
        scratch_types=[
            pltpu.VMEM((60 * 64,), jnp.float32),
            pltpu.VMEM((COLS_PER_TILE * 64,), jnp.float32),
            pltpu.VMEM((64,), jnp.float32),
            pltpu.VMEM((KMAX * 64,), jnp.float32),
            pltpu.VMEM((48,), jnp.float32),
            pltpu.VMEM((64,), jnp.int32),
            pltpu.VMEM((64,), jnp.float32),
            pltpu.SemaphoreType.DMA,
        ],
    )


def kernel(W1, b1, W2, b2, W3, b3):
    f32 = jnp.float32
    b1p = jnp.concatenate([b1, jnp.array([1.0, 0.0, 0.0, 0.0], f32)])
    w2t = jnp.concatenate(
        [W2.T, b2[:, None], jnp.zeros((60, 3), f32)], axis=1).reshape(-1)
    w3t = jnp.concatenate(
        [W3.T, b3[:, None], jnp.zeros((W3.shape[1], 3), f32)], axis=1)
    w3t = jnp.concatenate([w3t, jnp.zeros((NUM_TILES * COLS_PER_TILE - W3.shape[1], 64),
                                          f32)], axis=0).reshape(-1)
    gum = jnp.asarray(_GUMBEL_TILED)

    act_flat, sel_flat = _sc_kernel()(w2t, w3t, b1p, gum)
    actions = act_flat[:COUNT * BATCH].reshape(COUNT, BATCH).T
    selected = sel_flat[:COUNT * BATCH].reshape(COUNT, BATCH).T
    return (actions, selected)


# speedup vs baseline: 44.8706x; 1.0640x over previous
import functools

import numpy as np
import jax
import jax.numpy as jnp
from jax import lax
from jax.experimental import pallas as pl
from jax.experimental.pallas import tpu as pltpu
from jax.experimental.pallas import tpu_sc as plsc

COUNT = 255
BATCH = 8
NODES_PER_TILE = 8
COLS_PER_TILE = 40
NUM_CORES = 2
NUM_SUBCORES = 16
NUM_TILES = 32
TASKS = 2048
KMAX = 6


_ROT = [[13, 15, 26, 6], [17, 29, 16, 24]]


def _rotl(x, d):
    return ((x << np.uint32(d)) | (x >> np.uint32(32 - d))).astype(np.uint32)


def _threefry2x32(k1, k2, x0, x1):
    ks = [np.uint32(k1), np.uint32(k2), np.uint32(k1 ^ k2 ^ 0x1BD11BDA)]
    x = [(x0 + ks[0]).astype(np.uint32), (x1 + ks[1]).astype(np.uint32)]

    def rounds(x, rots):
        for r in rots:
            a = (x[0] + x[1]).astype(np.uint32)
            b = (_rotl(x[1], r) ^ a).astype(np.uint32)
            x = [a, b]
        return x

    for i, (inj0, inj1) in enumerate([(1, 2), (2, 0), (0, 1), (1, 2), (2, 0)]):
        x = rounds(x, _ROT[i % 2])
        x = [(x[0] + ks[inj0]).astype(np.uint32),
             (x[1] + ks[inj1] + np.uint32(i + 1)).astype(np.uint32)]
    return x[0], x[1]


def _fold_in(key, i):
    y0, y1 = _threefry2x32(key[0], key[1],
                           np.array([0], np.uint32), np.array([i], np.uint32))
    return (int(y0[0]), int(y1[0]))


def _gumbel(key, shape):
    n = int(np.prod(shape))
    y0, y1 = _threefry2x32(key[0], key[1],
                           np.zeros(n, np.uint32), np.arange(n, dtype=np.uint32))
    bits = y0 ^ y1
    u = ((bits >> np.uint32(9)) | np.uint32(0x3F800000)).view(np.float32) - np.float32(1.0)
    tiny = np.float32(np.finfo(np.float32).tiny)
    u = np.maximum(tiny, (u * (np.float32(1.0) - tiny) + tiny).astype(np.float32))
    return (-np.log(-np.log(u.astype(np.float64)))).astype(np.float32).reshape(shape)


def _build_gumbel_table():
    g = np.zeros((KMAX, TASKS), np.float32)
    for n in range(COUNT):
        arity = 6 if n % 2 == 0 else 4
        gn = _gumbel(_fold_in((0, 42), n), (BATCH, arity))
        g[:arity, n * BATCH:(n + 1) * BATCH] = gn.T
    return g


_GUMBEL = _build_gumbel_table()
_GUMBEL_TILED = np.ascontiguousarray(
    _GUMBEL.reshape(KMAX, 32, 64).transpose(1, 0, 2)).reshape(-1)


_NEG = -3.0e38


def _sc_body(w2t_hbm, w3w_hbm, b1_hbm, gum_hbm, act_hbm, sel_hbm,
             w2t_v, w3w_v, b1_v, gum_v, logits_v, act_v, sel_v,
             sem_a, sem_b, sem_c, sem_o):
    wid = lax.axis_index("s") * NUM_CORES + lax.axis_index("c")

    cp_b1 = pltpu.async_copy(b1_hbm, b1_v, sem_a)
    cp_w2 = pltpu.async_copy(w2t_hbm, w2t_v, sem_a)
    cp_w3 = pltpu.async_copy(
        w3w_hbm.at[pl.ds(wid * (61 * COLS_PER_TILE), 61 * COLS_PER_TILE)],
        w3w_v, sem_b)
    cp_gm = pltpu.async_copy(gum_hbm.at[pl.ds(wid * (KMAX * 64), KMAX * 64)],
                             gum_v, sem_c)
    cp_b1.wait()
    cp_w2.wait()

    h1 = [jnp.maximum(b1_v[pl.ds(16 * q, 16)], 0.0) for q in range(4)]
    h2 = []
    for j in range(60):
        acc = h1[0] * w2t_v[pl.ds(j * 64, 16)]
        for q in range(1, 4):
            acc = acc + h1[q] * w2t_v[pl.ds(j * 64 + 16 * q, 16)]
        h2.append(jnp.maximum(jnp.sum(acc), 0.0))

    cp_w3.wait()

    for off in (0, 16, 24):
        acc = w3w_v[pl.ds(60 * COLS_PER_TILE + off, 16)]
        for k in range(60):
            acc = acc + h2[k] * w3w_v[pl.ds(k * COLS_PER_TILE + off, 16)]
        e = jnp.exp(acc * 0.4)
        logits_v[pl.ds(off, 16)] = 2.5 * (1.0 - 2.0 / (e + 1.0))
    logits_v[pl.ds(COLS_PER_TILE, 16)] = jnp.full((16,), _NEG, jnp.float32)

    cp_gm.wait()

    lane = lax.iota(jnp.int32, 16)
    is_hi = lane >= 8
    b_of = (lane & 7) * 8 + (lane >> 3)
    for g in range(4):
        colstart = jnp.where(is_hi, 10 * g + 6, 10 * g)
        idx = ([colstart + k for k in range(4)]
               + [jnp.where(is_hi, COLS_PER_TILE + 7, 10 * g + k)
                  for k in (4, 5)])
        vk = [plsc.load_gather(logits_v, [idx[k]]) for k in range(KMAX)]
        gk = [gum_v[pl.ds(k * 64 + 16 * g, 16)] for k in range(KMAX)]
        m = vk[0]
        for k in range(1, KMAX):
            m = jnp.maximum(m, vk[k])
        sumexp = jnp.zeros((16,), jnp.float32)
        best_s = jnp.full((16,), _NEG, jnp.float32)
        best_k = jnp.zeros((16,), jnp.int32)
        best_e = jnp.zeros((16,), jnp.float32)
        for k in range(KMAX):
            e = jnp.exp(vk[k] - m)
            sumexp = sumexp + e
            s = vk[k] + gk[k]
            upd = s > best_s
            if k < KMAX - 1:
                best_s = jnp.where(upd, s, best_s)
            best_k = jnp.where(upd, jnp.full((16,), k, jnp.int32), best_k)
            best_e = jnp.where(upd, e, best_e)
        pos = b_of + 2 * g
        plsc.store_scatter(act_v, [pos], best_k)
        plsc.store_scatter(sel_v, [pos], best_e / sumexp)

    outs = []
    for b in range(8):
        dst = b * 256 + wid * 8
        outs.append(pltpu.async_copy(act_v.at[pl.ds(b * 8, 8)],
                                     act_hbm.at[pl.ds(dst, 8)], sem_o))
        outs.append(pltpu.async_copy(sel_v.at[pl.ds(b * 8, 8)],
                                     sel_hbm.at[pl.ds(dst, 8)], sem_o))
    for c in outs:
        c.wait()


@functools.cache
def _sc_kernel():
    return pl.kernel(
        _sc_body,
        out_type=(jax.ShapeDtypeStruct((TASKS,), jnp.int32),
                  jax.ShapeDtypeStruct((TASKS,), jnp.float32)),
        mesh=plsc.VectorSubcoreMesh(core_axis_name="c", subcore_axis_name="s",
                                    num_cores=NUM_CORES,
                                    num_subcores=NUM_SUBCORES),
        compiler_params=pltpu.CompilerParams(needs_layout_passes=False),
        scratch_types=[
            pltpu.VMEM((60 * 64,), jnp.float32),
            pltpu.VMEM((61 * COLS_PER_TILE,), jnp.float32),
            pltpu.VMEM((64,), jnp.float32),
            pltpu.VMEM((KMAX * 64,), jnp.float32),
            pltpu.VMEM((COLS_PER_TILE + 16,), jnp.float32),
            pltpu.VMEM((64,), jnp.int32),
            pltpu.VMEM((64,), jnp.float32),
            pltpu.SemaphoreType.DMA,
            pltpu.SemaphoreType.DMA,
            pltpu.SemaphoreType.DMA,
            pltpu.SemaphoreType.DMA,
        ],
    )


def kernel(W1, b1, W2, b2, W3, b3):
    f32 = jnp.float32
    b1p = jnp.concatenate([b1, jnp.array([1.0, 0.0, 0.0, 0.0], f32)])
    w2t = jnp.concatenate(
        [W2.T, b2[:, None], jnp.zeros((60, 3), f32)], axis=1).reshape(-1)
    ncol = W3.shape[1]
    w3a = jnp.concatenate([W3, b3[None, :]], axis=0)
    w3a = jnp.concatenate(
        [w3a, jnp.zeros((61, NUM_TILES * COLS_PER_TILE - ncol), f32)], axis=1)
    w3w = w3a.reshape(61, NUM_TILES, COLS_PER_TILE).transpose(1, 0, 2).reshape(-1)
    gum = jnp.asarray(_GUMBEL_TILED)

    act_flat, sel_flat = _sc_kernel()(w2t, w3w, b1p, gum)
    actions = act_flat.reshape(BATCH, 256)[:, :COUNT]
    selected = sel_flat.reshape(BATCH, 256)[:, :COUNT]
    return (actions, selected)


# speedup vs baseline: 47.1953x; 1.0518x over previous
import functools

import numpy as np
import jax
import jax.numpy as jnp
from jax import lax
from jax.experimental import pallas as pl
from jax.experimental.pallas import tpu as pltpu
from jax.experimental.pallas import tpu_sc as plsc

COUNT = 255
BATCH = 8
NODES_PER_TILE = 8
COLS_PER_TILE = 40
NUM_CORES = 2
NUM_SUBCORES = 16
NUM_TILES = 32
TASKS = 2048
KMAX = 6


_ROT = [[13, 15, 26, 6], [17, 29, 16, 24]]


def _rotl(x, d):
    return ((x << np.uint32(d)) | (x >> np.uint32(32 - d))).astype(np.uint32)


def _threefry2x32(k1, k2, x0, x1):
    ks = [np.uint32(k1), np.uint32(k2), np.uint32(k1 ^ k2 ^ 0x1BD11BDA)]
    x = [(x0 + ks[0]).astype(np.uint32), (x1 + ks[1]).astype(np.uint32)]

    def rounds(x, rots):
        for r in rots:
            a = (x[0] + x[1]).astype(np.uint32)
            b = (_rotl(x[1], r) ^ a).astype(np.uint32)
            x = [a, b]
        return x

    for i, (inj0, inj1) in enumerate([(1, 2), (2, 0), (0, 1), (1, 2), (2, 0)]):
        x = rounds(x, _ROT[i % 2])
        x = [(x[0] + ks[inj0]).astype(np.uint32),
             (x[1] + ks[inj1] + np.uint32(i + 1)).astype(np.uint32)]
    return x[0], x[1]


def _fold_in(key, i):
    y0, y1 = _threefry2x32(key[0], key[1],
                           np.array([0], np.uint32), np.array([i], np.uint32))
    return (int(y0[0]), int(y1[0]))


def _gumbel(key, shape):
    n = int(np.prod(shape))
    y0, y1 = _threefry2x32(key[0], key[1],
                           np.zeros(n, np.uint32), np.arange(n, dtype=np.uint32))
    bits = y0 ^ y1
    u = ((bits >> np.uint32(9)) | np.uint32(0x3F800000)).view(np.float32) - np.float32(1.0)
    tiny = np.float32(np.finfo(np.float32).tiny)
    u = np.maximum(tiny, (u * (np.float32(1.0) - tiny) + tiny).astype(np.float32))
    return (-np.log(-np.log(u.astype(np.float64)))).astype(np.float32).reshape(shape)


def _build_gumbel_table():
    g = np.zeros((KMAX, TASKS), np.float32)
    for n in range(COUNT):
        arity = 6 if n % 2 == 0 else 4
        gn = _gumbel(_fold_in((0, 42), n), (BATCH, arity))
        g[:arity, n * BATCH:(n + 1) * BATCH] = gn.T
    return g


_GUMBEL = _build_gumbel_table()
_GUMBEL_TILED = np.ascontiguousarray(
    _GUMBEL.reshape(KMAX, 32, 64).transpose(1, 0, 2)).reshape(-1)


_NEG = -3.0e38


def _sc_body(w2_hbm, w3_hbm, b12_hbm, gum_hbm, act_hbm, sel_hbm,
             w2_v, w3_v, b12_v, gum_v, logits_v, act_v, sel_v,
             sem_a, sem_b, sem_c, sem_o):
    wid = lax.axis_index("s") * NUM_CORES + lax.axis_index("c")

    cp_b1 = pltpu.async_copy(b12_hbm, b12_v, sem_a)
    cp_w2 = pltpu.async_copy(w2_hbm, w2_v, sem_a)
    col0 = wid * COLS_PER_TILE
    cp_w3 = [pltpu.async_copy(
        w3_hbm.at[pl.ds(k * NUM_TILES * COLS_PER_TILE + col0, COLS_PER_TILE)],
        w3_v.at[k], sem_b) for k in range(61)]
    cp_gm = pltpu.async_copy(gum_hbm.at[pl.ds(wid * (KMAX * 64), KMAX * 64)],
                             gum_v, sem_c)
    cp_b1.wait()
    cp_w2.wait()

    QA = (0, 16, 32, 44)
    h1v = [b12_v[pl.ds(16 * q, 16)] for q in range(4)]
    acc = [b12_v[pl.ds(64 + off, 16)] for off in QA]
    for k in range(60):
        s = jnp.maximum(h1v[k // 16][k % 16], 0.0)
        for qi, off in enumerate(QA):
            acc[qi] = acc[qi] + s * w2_v[k, pl.ds(off, 16)]
    h2v = [jnp.maximum(a, 0.0) for a in acc]
    h2 = [h2v[k // 16][k % 16] if k < 48 else h2v[3][k - 44]
          for k in range(60)]

    for c in cp_w3:
        c.wait()

    for off in (0, 16, 24):
        accb = w3_v[60, pl.ds(off, 16)]
        for k in range(60):
            accb = accb + h2[k] * w3_v[k, pl.ds(off, 16)]
        e = jnp.exp(accb * 0.4)
        logits_v[pl.ds(off, 16)] = 2.5 * (1.0 - 2.0 / (e + 1.0))
    logits_v[pl.ds(COLS_PER_TILE, 16)] = jnp.full((16,), _NEG, jnp.float32)

    cp_gm.wait()

    lane = lax.iota(jnp.int32, 16)
    is_hi = lane >= 8
    b_of = (lane & 7) * 8 + (lane >> 3)
    for g in range(4):
        colstart = jnp.where(is_hi, 10 * g + 6, 10 * g)
        idx = ([colstart + k for k in range(4)]
               + [jnp.where(is_hi, COLS_PER_TILE + 7, 10 * g + k)
                  for k in (4, 5)])
        vk = [plsc.load_gather(logits_v, [idx[k]]) for k in range(KMAX)]
        gk = [gum_v[pl.ds(k * 64 + 16 * g, 16)] for k in range(KMAX)]
        m = vk[0]
        for k in range(1, KMAX):
            m = jnp.maximum(m, vk[k])
        sumexp = jnp.zeros((16,), jnp.float32)
        best_s = jnp.full((16,), _NEG, jnp.float32)
        best_k = jnp.zeros((16,), jnp.int32)
        best_e = jnp.zeros((16,), jnp.float32)
        for k in range(KMAX):
            e = jnp.exp(vk[k] - m)
            sumexp = sumexp + e
            s = vk[k] + gk[k]
            upd = s > best_s
            if k < KMAX - 1:
                best_s = jnp.where(upd, s, best_s)
            best_k = jnp.where(upd, jnp.full((16,), k, jnp.int32), best_k)
            best_e = jnp.where(upd, e, best_e)
        pos = b_of + 2 * g
        plsc.store_scatter(act_v, [pos], best_k)
        plsc.store_scatter(sel_v, [pos], best_e / sumexp)

    outs = []
    for b in range(8):
        outs.append(pltpu.async_copy(act_v.at[pl.ds(b * 8, 8)],
                                     act_hbm.at[b, pl.ds(wid * 8, 8)], sem_o))
        outs.append(pltpu.async_copy(sel_v.at[pl.ds(b * 8, 8)],
                                     sel_hbm.at[b, pl.ds(wid * 8, 8)], sem_o))
    for c in outs:
        c.wait()


@functools.cache
def _sc_kernel():
    return pl.kernel(
        _sc_body,
        out_type=(jax.ShapeDtypeStruct((BATCH, 256), jnp.int32),
                  jax.ShapeDtypeStruct((BATCH, 256), jnp.float32)),
        mesh=plsc.VectorSubcoreMesh(core_axis_name="c", subcore_axis_name="s",
                                    num_cores=NUM_CORES,
                                    num_subcores=NUM_SUBCORES),
        compiler_params=pltpu.CompilerParams(needs_layout_passes=False),
        scratch_types=[
            pltpu.VMEM((60, 60), jnp.float32),
            pltpu.VMEM((61, COLS_PER_TILE), jnp.float32),
            pltpu.VMEM((128,), jnp.float32),
            pltpu.VMEM((KMAX * 64,), jnp.float32),
            pltpu.VMEM((COLS_PER_TILE + 16,), jnp.float32),
            pltpu.VMEM((64,), jnp.int32),
            pltpu.VMEM((64,), jnp.float32),
            pltpu.SemaphoreType.DMA,
            pltpu.SemaphoreType.DMA,
            pltpu.SemaphoreType.DMA,
            pltpu.SemaphoreType.DMA,
        ],
    )


def kernel(W1, b1, W2, b2, W3, b3):
    f32 = jnp.float32
    zeros4 = jnp.zeros((4,), f32)
    b12 = jnp.concatenate([b1, zeros4, b2, zeros4])
    ncol = W3.shape[1]
    w3a = jnp.concatenate([W3, b3[None, :]], axis=0)
    w3a = jnp.concatenate(
        [w3a, jnp.zeros((61, NUM_TILES * COLS_PER_TILE - ncol), f32)],
        axis=1).reshape(-1)
    gum = jnp.asarray(_GUMBEL_TILED)

    act2d, sel2d = _sc_kernel()(W2, w3a, b12, gum)
    actions = act2d[:, :COUNT]
    selected = sel2d[:, :COUNT]
    return (actions, selected)


# speedup vs baseline: 48.0455x; 1.0180x over previous
import functools

import numpy as np
import jax
import jax.numpy as jnp
from jax import lax
from jax.experimental import pallas as pl
from jax.experimental.pallas import tpu as pltpu
from jax.experimental.pallas import tpu_sc as plsc

COUNT = 255
BATCH = 8
NODES_PER_TILE = 8
COLS_PER_TILE = 40
NUM_CORES = 2
NUM_SUBCORES = 16
NUM_TILES = 32
TASKS = 2048
KMAX = 6


_ROT = [[13, 15, 26, 6], [17, 29, 16, 24]]


def _rotl(x, d):
    return ((x << np.uint32(d)) | (x >> np.uint32(32 - d))).astype(np.uint32)


def _threefry2x32(k1, k2, x0, x1):
    ks = [np.uint32(k1), np.uint32(k2), np.uint32(k1 ^ k2 ^ 0x1BD11BDA)]
    x = [(x0 + ks[0]).astype(np.uint32), (x1 + ks[1]).astype(np.uint32)]

    def rounds(x, rots):
        for r in rots:
            a = (x[0] + x[1]).astype(np.uint32)
            b = (_rotl(x[1], r) ^ a).astype(np.uint32)
            x = [a, b]
        return x

    for i, (inj0, inj1) in enumerate([(1, 2), (2, 0), (0, 1), (1, 2), (2, 0)]):
        x = rounds(x, _ROT[i % 2])
        x = [(x[0] + ks[inj0]).astype(np.uint32),
             (x[1] + ks[inj1] + np.uint32(i + 1)).astype(np.uint32)]
    return x[0], x[1]


def _fold_in(key, i):
    y0, y1 = _threefry2x32(key[0], key[1],
                           np.array([0], np.uint32), np.array([i], np.uint32))
    return (int(y0[0]), int(y1[0]))


def _gumbel(key, shape):
    n = int(np.prod(shape))
    y0, y1 = _threefry2x32(key[0], key[1],
                           np.zeros(n, np.uint32), np.arange(n, dtype=np.uint32))
    bits = y0 ^ y1
    u = ((bits >> np.uint32(9)) | np.uint32(0x3F800000)).view(np.float32) - np.float32(1.0)
    tiny = np.float32(np.finfo(np.float32).tiny)
    u = np.maximum(tiny, (u * (np.float32(1.0) - tiny) + tiny).astype(np.float32))
    return (-np.log(-np.log(u.astype(np.float64)))).astype(np.float32).reshape(shape)


def _build_gumbel_table():
    g = np.zeros((KMAX, TASKS), np.float32)
    for n in range(COUNT):
        arity = 6 if n % 2 == 0 else 4
        gn = _gumbel(_fold_in((0, 42), n), (BATCH, arity))
        g[:arity, n * BATCH:(n + 1) * BATCH] = gn.T
    return g


_GUMBEL = _build_gumbel_table()
_GUMBEL_TILED = np.ascontiguousarray(
    _GUMBEL.reshape(KMAX, 32, 64).transpose(1, 0, 2)).reshape(-1)


_NEG = -3.0e38


def _sc_body(w2_hbm, w3_hbm, bc_hbm, gum_hbm, act_hbm, sel_hbm,
             w2_v, w3_v, b12_v, b3_v, gum_v, logits_v, act_v, sel_v,
             sem_a, sem_b, sem_c, sem_o):
    wid = lax.axis_index("s") * NUM_CORES + lax.axis_index("c")

    col0 = wid * COLS_PER_TILE
    start = jnp.minimum((col0 // 128) * 128, 1280 - 256)
    local = col0 - start

    cp_b1 = pltpu.async_copy(bc_hbm.at[pl.ds(0, 128)], b12_v, sem_a)
    cp_w2 = pltpu.async_copy(w2_hbm, w2_v, sem_a)
    cp_w3 = pltpu.async_copy(w3_hbm.at[:, pl.ds(start, 256)], w3_v, sem_b)
    cp_b3 = pltpu.async_copy(bc_hbm.at[pl.ds(120 + col0, 48)], b3_v, sem_b)
    cp_gm = pltpu.async_copy(gum_hbm.at[pl.ds(wid * (KMAX * 64), KMAX * 64)],
                             gum_v, sem_c)
    cp_b1.wait()
    cp_w2.wait()

    QA = (0, 16, 32, 44)
    h1v = [b12_v[pl.ds(off, 16)] for off in QA]
    acc = [b12_v[pl.ds(60 + off, 16)] for off in QA]
    for k in range(60):
        s = (jnp.maximum(h1v[k // 16][k % 16], 0.0) if k < 48
             else jnp.maximum(h1v[3][k - 44], 0.0))
        for qi, off in enumerate(QA):
            acc[qi] = acc[qi] + s * w2_v[k, pl.ds(off, 16)]
    h2v = [jnp.maximum(a, 0.0) for a in acc]
    h2 = [h2v[k // 16][k % 16] if k < 48 else h2v[3][k - 44]
          for k in range(60)]

    cp_w3.wait()
    cp_b3.wait()

    for off in (0, 16, 24):
        accb = b3_v[pl.ds(off, 16)]
        for k in range(60):
            accb = accb + h2[k] * w3_v[k, pl.ds(local + off, 16)]
        e = jnp.exp(accb * 0.4)
        logits_v[pl.ds(off, 16)] = 2.5 * (1.0 - 2.0 / (e + 1.0))
    logits_v[pl.ds(COLS_PER_TILE, 16)] = jnp.full((16,), _NEG, jnp.float32)

    cp_gm.wait()

    lane = lax.iota(jnp.int32, 16)
    is_hi = lane >= 8
    b_of = (lane & 7) * 8 + (lane >> 3)
    for g in range(4):
        colstart = jnp.where(is_hi, 10 * g + 6, 10 * g)
        idx = ([colstart + k for k in range(4)]
               + [jnp.where(is_hi, COLS_PER_TILE + 7, 10 * g + k)
                  for k in (4, 5)])
        vk = [plsc.load_gather(logits_v, [idx[k]]) for k in range(KMAX)]
        gk = [gum_v[pl.ds(k * 64 + 16 * g, 16)] for k in range(KMAX)]
        m = vk[0]
        for k in range(1, KMAX):
            m = jnp.maximum(m, vk[k])
        sumexp = jnp.zeros((16,), jnp.float32)
        best_s = jnp.full((16,), _NEG, jnp.float32)
        best_k = jnp.zeros((16,), jnp.int32)
        best_e = jnp.zeros((16,), jnp.float32)
        for k in range(KMAX):
            e = jnp.exp(vk[k] - m)
            sumexp = sumexp + e
            s = vk[k] + gk[k]
            upd = s > best_s
            if k < KMAX - 1:
                best_s = jnp.where(upd, s, best_s)
            best_k = jnp.where(upd, jnp.full((16,), k, jnp.int32), best_k)
            best_e = jnp.where(upd, e, best_e)
        pos = b_of + 2 * g
        plsc.store_scatter(act_v, [pos], best_k)
        plsc.store_scatter(sel_v, [pos], best_e / sumexp)

    outs = []
    for b in range(8):
        outs.append(pltpu.async_copy(act_v.at[pl.ds(b * 8, 8)],
                                     act_hbm.at[b, pl.ds(wid * 8, 8)], sem_o))
        outs.append(pltpu.async_copy(sel_v.at[pl.ds(b * 8, 8)],
                                     sel_hbm.at[b, pl.ds(wid * 8, 8)], sem_o))
    for c in outs:
        c.wait()


@functools.cache
def _sc_kernel():
    return pl.kernel(
        _sc_body,
        out_type=(jax.ShapeDtypeStruct((BATCH, 256), jnp.int32),
                  jax.ShapeDtypeStruct((BATCH, 256), jnp.float32)),
        mesh=plsc.VectorSubcoreMesh(core_axis_name="c", subcore_axis_name="s",
                                    num_cores=NUM_CORES,
                                    num_subcores=NUM_SUBCORES),
        compiler_params=pltpu.CompilerParams(needs_layout_passes=False),
        scratch_types=[
            pltpu.VMEM((60, 60), jnp.float32),
            pltpu.VMEM((60, 256), jnp.float32),
            pltpu.VMEM((128,), jnp.float32),
            pltpu.VMEM((48,), jnp.float32),
            pltpu.VMEM((KMAX * 64,), jnp.float32),
            pltpu.VMEM((COLS_PER_TILE + 16,), jnp.float32),
            pltpu.VMEM((64,), jnp.int32),
            pltpu.VMEM((64,), jnp.float32),
            pltpu.SemaphoreType.DMA,
            pltpu.SemaphoreType.DMA,
            pltpu.SemaphoreType.DMA,
            pltpu.SemaphoreType.DMA,
        ],
    )


def kernel(W1, b1, W2, b2, W3, b3):
    f32 = jnp.float32
    bcat = jnp.concatenate([b1, b2, b3, jnp.zeros((12,), f32)])
    w3p = jnp.pad(W3, ((0, 0), (0, NUM_TILES * COLS_PER_TILE - W3.shape[1])))
    gum = jnp.asarray(_GUMBEL_TILED)

    act2d, sel2d = _sc_kernel()(W2, w3p, bcat, gum)
    actions = act2d[:, :COUNT]
    selected = sel2d[:, :COUNT]
    return (actions, selected)


# speedup vs baseline: 51.3052x; 1.0678x over previous
import functools

import numpy as np
import jax
import jax.numpy as jnp
from jax import lax
from jax.experimental import pallas as pl
from jax.experimental.pallas import tpu as pltpu
from jax.experimental.pallas import tpu_sc as plsc

COUNT = 255
BATCH = 8
NODES_PER_TILE = 16
COLS_PER_TILE = 80
NUM_CORES = 1
NUM_SUBCORES = 16
NUM_TILES = NUM_CORES * NUM_SUBCORES
TASKS = 2048
KMAX = 6
GROUPS = NODES_PER_TILE // 2
GSTRIDE = NODES_PER_TILE * BATCH
TPT = TASKS // NUM_TILES


_ROT = [[13, 15, 26, 6], [17, 29, 16, 24]]


def _rotl(x, d):
    return ((x << np.uint32(d)) | (x >> np.uint32(32 - d))).astype(np.uint32)


def _threefry2x32(k1, k2, x0, x1):
    ks = [np.uint32(k1), np.uint32(k2), np.uint32(k1 ^ k2 ^ 0x1BD11BDA)]
    x = [(x0 + ks[0]).astype(np.uint32), (x1 + ks[1]).astype(np.uint32)]

    def rounds(x, rots):
        for r in rots:
            a = (x[0] + x[1]).astype(np.uint32)
            b = (_rotl(x[1], r) ^ a).astype(np.uint32)
            x = [a, b]
        return x

    for i, (inj0, inj1) in enumerate([(1, 2), (2, 0), (0, 1), (1, 2), (2, 0)]):
        x = rounds(x, _ROT[i % 2])
        x = [(x[0] + ks[inj0]).astype(np.uint32),
             (x[1] + ks[inj1] + np.uint32(i + 1)).astype(np.uint32)]
    return x[0], x[1]


def _fold_in(key, i):
    y0, y1 = _threefry2x32(key[0], key[1],
                           np.array([0], np.uint32), np.array([i], np.uint32))
    return (int(y0[0]), int(y1[0]))


def _gumbel(key, shape):
    n = int(np.prod(shape))
    y0, y1 = _threefry2x32(key[0], key[1],
                           np.zeros(n, np.uint32), np.arange(n, dtype=np.uint32))
    bits = y0 ^ y1
    u = ((bits >> np.uint32(9)) | np.uint32(0x3F800000)).view(np.float32) - np.float32(1.0)
    tiny = np.float32(np.finfo(np.float32).tiny)
    u = np.maximum(tiny, (u * (np.float32(1.0) - tiny) + tiny).astype(np.float32))
    return (-np.log(-np.log(u.astype(np.float64)))).astype(np.float32).reshape(shape)


def _build_gumbel_table():
    g = np.zeros((KMAX, TASKS), np.float32)
    for n in range(COUNT):
        arity = 6 if n % 2 == 0 else 4
        gn = _gumbel(_fold_in((0, 42), n), (BATCH, arity))
        g[:arity, n * BATCH:(n + 1) * BATCH] = gn.T
    return g


_GUMBEL = _build_gumbel_table()
_GUMBEL_TILED = np.ascontiguousarray(
    _GUMBEL.reshape(KMAX, NUM_TILES, GSTRIDE).transpose(1, 0, 2)).reshape(-1)


_NEG = -3.0e38


def _sc_body(w2_hbm, w3_hbm, bc_hbm, gum_hbm, act_hbm, sel_hbm,
             w2_v, w3_v, b12_v, b3_v, gum_v, logits_v, act_v, sel_v,
             sem_a, sem_b, sem_c, sem_o):
    wid = lax.axis_index("s") * NUM_CORES + lax.axis_index("c")

    col0 = wid * COLS_PER_TILE
    start = jnp.minimum((col0 // 128) * 128, 1280 - 256)
    local = col0 - start

    cp_b1 = pltpu.async_copy(bc_hbm.at[pl.ds(0, 128)], b12_v, sem_a)
    cp_w2 = pltpu.async_copy(w2_hbm, w2_v, sem_a)
    cp_w3 = pltpu.async_copy(w3_hbm.at[:, pl.ds(start, 256)], w3_v, sem_b)
    cp_b3 = pltpu.async_copy(bc_hbm.at[pl.ds(120 + col0, COLS_PER_TILE)],
                             b3_v, sem_b)
    cp_gm = pltpu.async_copy(
        gum_hbm.at[pl.ds(wid * (KMAX * GSTRIDE), KMAX * GSTRIDE)],
        gum_v, sem_c)
    cp_b1.wait()
    cp_w2.wait()

    QA = (0, 16, 32, 44)
    h1v = [b12_v[pl.ds(off, 16)] for off in QA]
    acc = [b12_v[pl.ds(60 + off, 16)] for off in QA]
    for k in range(60):
        s = (jnp.maximum(h1v[k // 16][k % 16], 0.0) if k < 48
             else jnp.maximum(h1v[3][k - 44], 0.0))
        for qi, off in enumerate(QA):
            acc[qi] = acc[qi] + s * w2_v[k, pl.ds(off, 16)]
    h2v = [jnp.maximum(a, 0.0) for a in acc]
    h2 = [h2v[k // 16][k % 16] if k < 48 else h2v[3][k - 44]
          for k in range(60)]

    cp_w3.wait()
    cp_b3.wait()

    for off in range(0, COLS_PER_TILE, 16):
        accb = b3_v[pl.ds(off, 16)]
        for k in range(60):
            accb = accb + h2[k] * w3_v[k, pl.ds(local + off, 16)]
        e = jnp.exp(accb * 0.4)
        logits_v[pl.ds(off, 16)] = 2.5 * (1.0 - 2.0 / (e + 1.0))
    logits_v[pl.ds(COLS_PER_TILE, 16)] = jnp.full((16,), _NEG, jnp.float32)

    cp_gm.wait()

    lane = lax.iota(jnp.int32, 16)
    is_hi = lane >= 8
    b_of = (lane & 7) * NODES_PER_TILE + (lane >> 3)
    for g in range(GROUPS):
        colstart = jnp.where(is_hi, 10 * g + 6, 10 * g)
        idx = ([colstart + k for k in range(4)]
               + [jnp.where(is_hi, COLS_PER_TILE + 7, 10 * g + k)
                  for k in (4, 5)])
        vk = [plsc.load_gather(logits_v, [idx[k]]) for k in range(KMAX)]
        gk = [gum_v[pl.ds(k * GSTRIDE + 16 * g, 16)] for k in range(KMAX)]
        m = vk[0]
        for k in range(1, KMAX):
            m = jnp.maximum(m, vk[k])
        sumexp = jnp.zeros((16,), jnp.float32)
        best_s = jnp.full((16,), _NEG, jnp.float32)
        best_k = jnp.zeros((16,), jnp.int32)
        best_e = jnp.zeros((16,), jnp.float32)
        for k in range(KMAX):
            e = jnp.exp(vk[k] - m)
            sumexp = sumexp + e
            s = vk[k] + gk[k]
            upd = s > best_s
            if k < KMAX - 1:
                best_s = jnp.where(upd, s, best_s)
            best_k = jnp.where(upd, jnp.full((16,), k, jnp.int32), best_k)
            best_e = jnp.where(upd, e, best_e)
        pos = b_of + 2 * g
        plsc.store_scatter(act_v, [pos], best_k)
        plsc.store_scatter(sel_v, [pos], best_e / sumexp)

    npt = NODES_PER_TILE
    outs = []
    for b in range(8):
        outs.append(pltpu.async_copy(act_v.at[pl.ds(b * npt, npt)],
                                     act_hbm.at[b, pl.ds(wid * npt, npt)], sem_o))
        outs.append(pltpu.async_copy(sel_v.at[pl.ds(b * npt, npt)],
                                     sel_hbm.at[b, pl.ds(wid * npt, npt)], sem_o))
    for c in outs:
        c.wait()


@functools.cache
def _sc_kernel():
    return pl.kernel(
        _sc_body,
        out_type=(jax.ShapeDtypeStruct((BATCH, 256), jnp.int32),
                  jax.ShapeDtypeStruct((BATCH, 256), jnp.float32)),
        mesh=plsc.VectorSubcoreMesh(core_axis_name="c", subcore_axis_name="s",
                                    num_cores=NUM_CORES,
                                    num_subcores=NUM_SUBCORES),
        compiler_params=pltpu.CompilerParams(needs_layout_passes=False),
        scratch_types=[
            pltpu.VMEM((60, 60), jnp.float32),
            pltpu.VMEM((60, 256), jnp.float32),
            pltpu.VMEM((128,), jnp.float32),
            pltpu.VMEM((COLS_PER_TILE,), jnp.float32),
            pltpu.VMEM((KMAX * GSTRIDE,), jnp.float32),
            pltpu.VMEM((COLS_PER_TILE + 16,), jnp.float32),
            pltpu.VMEM((TPT,), jnp.int32),
            pltpu.VMEM((TPT,), jnp.float32),
            pltpu.SemaphoreType.DMA,
            pltpu.SemaphoreType.DMA,
            pltpu.SemaphoreType.DMA,
            pltpu.SemaphoreType.DMA,
        ],
    )


def kernel(W1, b1, W2, b2, W3, b3):
    f32 = jnp.float32
    bcat = jnp.concatenate([b1, b2, b3, jnp.zeros((20,), f32)])
    w3p = jnp.pad(W3, ((0, 0), (0, 1280 - W3.shape[1])))
    gum = jnp.asarray(_GUMBEL_TILED)

    act2d, sel2d = _sc_kernel()(W2, w3p, bcat, gum)
    actions = act2d[:, :COUNT]
    selected = sel2d[:, :COUNT]
    return (actions, selected)
